# baseline (speedup 1.0000x reference)
import jax
import jax.numpy as jnp
from jax import lax
from jax.experimental import pallas as pl
from jax.experimental.pallas import tpu as pltpu
from jax.experimental.pallas import tpu_sc as plsc

N = 10000
E = 160000
C_IN = 256
C_OUT = 256
HALF = 128
NS = 16
NC = 2
CH = 624
TAIL = N - NS * CH

NP = 10240
CH1 = NP // NS

B1 = 128
EPAD = 163840
ERC = EPAD // B1
NB1 = ERC // (NC * NS)
B2 = 128
NB2 = EPAD // (NS * B2)

def _make_mesh():
    return plsc.VectorSubcoreMesh(
        core_axis_name="c", subcore_axis_name="s", num_cores=NC, num_subcores=NS
    )


def _deg_body(dst2d_hbm, degp_hbm, dacc, widx, ones_v, zbuf):
    c = lax.axis_index("c")
    s = lax.axis_index("s")
    ones = jnp.ones((16,), jnp.float32)
    zeros = jnp.zeros((16,), jnp.float32)

    @pl.loop(0, B1 // 16)
    def _(i):
        ones_v[pl.ds(i * 16, 16)] = ones

    @pl.loop(0, CH1 // 16)
    def _(i):
        zbuf[pl.ds(i * 16, 16)] = zeros

    w = c * NS + s
    pltpu.sync_copy(dst2d_hbm.at[pl.ds(w * NB1, NB1)], widx)
    r0 = s * CH1
    pltpu.sync_copy(zbuf, dacc.at[pl.ds(r0, CH1)])
    plsc.subcore_barrier()

    @pl.loop(0, NB1)
    def _(g):
        pltpu.sync_copy(ones_v, dacc.at[widx.at[g]], add=True)

    plsc.subcore_barrier()
    pltpu.sync_copy(dacc.at[pl.ds(r0, CH1)], degp_hbm.at[pl.ds(c * NP + r0, CH1)])


def _make_deg():
    return pl.kernel(
        _deg_body,
        out_type=jax.ShapeDtypeStruct((NC * NP,), jnp.float32),
        mesh=_make_mesh(),
        scratch_types=[
            pltpu.VMEM_SHARED((NP,), jnp.float32),
            pltpu.VMEM((NB1, B1), jnp.int32),
            pltpu.VMEM((B1,), jnp.float32),
            pltpu.VMEM((CH1,), jnp.float32),
        ],
    )


def _agg_body(src2d_hbm, dst2d_hbm, hs2_hbm, z128_hbm, acc2_hbm, asp,
              sidx, didx0, didx1, rows0, rows1, sem0, sem1, ssem0, ssem1):
    c = lax.axis_index("c")
    s = lax.axis_index("s")
    r0 = s * CH
    pltpu.sync_copy(z128_hbm.at[pl.ds(r0, CH)], asp.at[pl.ds(r0, CH)])

    @pl.when(s == 0)
    def _():
        pltpu.sync_copy(z128_hbm.at[pl.ds(NS * CH, TAIL)], asp.at[pl.ds(NS * CH, TAIL)])

    pltpu.sync_copy(src2d_hbm.at[pl.ds(c * ERC + s * NB2, NB2)], sidx)
    plsc.subcore_barrier()
    rows = (rows0, rows1)
    didx = (didx0, didx1)
    sems = (sem0, sem1)
    ssems = (ssem0, ssem1)
    drow = s * NB2

    pltpu.sync_copy(dst2d_hbm.at[drow], didx0)
    pltpu.async_copy(hs2_hbm.at[sidx.at[0]], rows0, sem0)

    @pl.loop(0, NB2 // 2)
    def _(h):
        for b in range(2):
            g = 2 * h + b
            nxt = g + 1
            nb = 1 - b

            @pl.when(nxt < NB2)
            def _():
                @pl.when(g >= 1)
                def _():
                    pltpu.make_async_copy(rows[nb], asp.at[didx[nb]], ssems[nb]).wait()

                pltpu.sync_copy(dst2d_hbm.at[drow + nxt], didx[nb])
                pltpu.async_copy(hs2_hbm.at[sidx.at[nxt]], rows[nb], sems[nb])

            pltpu.make_async_copy(hs2_hbm.at[sidx.at[0]], rows[b], sems[b]).wait()
            pltpu.async_copy(rows[b], asp.at[didx[b]], ssems[b], add=True)

    pltpu.make_async_copy(rows0, asp.at[didx0], ssem0).wait()
    pltpu.make_async_copy(rows1, asp.at[didx1], ssem1).wait()
    plsc.subcore_barrier()
    for rep in range(2):
        g = s + NS * rep

        @pl.when(g < NBM)
        def _():
            pltpu.sync_copy(
                asp.at[pl.ds(g * BM, BM)],
                acc2_hbm.at[pl.ds(g * 2 * BM + c * BM, BM)],
            )


def _make_agg():
    return pl.kernel(
        _agg_body,
        out_type=jax.ShapeDtypeStruct((NC * N, HALF), jnp.float32),
        mesh=_make_mesh(),
        scratch_types=[
            pltpu.VMEM_SHARED((N, HALF), jnp.float32),
            pltpu.VMEM((NB2, B2), jnp.int32),
            pltpu.VMEM((B2,), jnp.int32),
            pltpu.VMEM((B2,), jnp.int32),
            pltpu.VMEM((B2, HALF), jnp.float32),
            pltpu.VMEM((B2, HALF), jnp.float32),
            pltpu.SemaphoreType.DMA,
            pltpu.SemaphoreType.DMA,
            pltpu.SemaphoreType.DMA,
            pltpu.SemaphoreType.DMA,
        ],
    )

BM = 400
NBM = N // BM


def _dinv_col(eye_b, d0_b, d1_b, i):
    d0 = d0_b[pl.ds(i, 1), :]
    d1 = d1_b[pl.ds(i, 1), :]
    dinv_row = lax.rsqrt(1.0 + d0 + d1)
    return lax.dot_general(eye_b[...], dinv_row, (((1,), (1,)), ((), ())),
                           preferred_element_type=jnp.float32)


def _mm_body(x_b, w_b, eye_b, d0_b, d1_b, o_b):
    i = pl.program_id(0)

    @pl.when(i < NBM)
    def _():
        dinv = _dinv_col(eye_b, d0_b, d1_b, jnp.minimum(i, NBM - 1))
        h = jnp.dot(x_b[...], w_b[...], preferred_element_type=jnp.float32)
        o_b[pl.ds(0, BM), :] = h[:, :HALF] * dinv
        o_b[pl.ds(BM, BM), :] = h[:, HALF:] * dinv

    @pl.when(i == NBM)
    def _():
        o_b[...] = jnp.zeros((2 * BM, HALF), jnp.float32)


def _make_mm():
    return pl.pallas_call(
        _mm_body,
        grid=(NBM + 1,),
        in_specs=[
            pl.BlockSpec((BM, C_IN), lambda i: (jnp.minimum(i, NBM - 1), 0)),
            pl.BlockSpec((C_IN, C_OUT), lambda i: (0, 0)),
            pl.BlockSpec((BM, BM), lambda i: (0, 0)),
            pl.BlockSpec((NBM, BM), lambda i: (0, 0)),
            pl.BlockSpec((NBM, BM), lambda i: (0, 0)),
        ],
        out_specs=pl.BlockSpec((2 * BM, HALF), lambda i: (i, 0)),
        out_shape=jax.ShapeDtypeStruct((NC * N + 2 * BM, HALF), jnp.float32),
    )


def _fin_body(a_b, h_b, eye_b, d0_b, d1_b, b_b, o_b):
    dinv = _dinv_col(eye_b, d0_b, d1_b, pl.program_id(0))
    t0 = (a_b[pl.ds(0, BM), :] + h_b[pl.ds(0, BM), :]) * dinv
    t1 = (a_b[pl.ds(BM, BM), :] + h_b[pl.ds(BM, BM), :]) * dinv
    o_b[...] = jnp.maximum(jnp.concatenate([t0, t1], axis=1) + b_b[...], 0.0)


def _make_fin():
    return pl.pallas_call(
        _fin_body,
        grid=(NBM,),
        in_specs=[
            pl.BlockSpec((2 * BM, HALF), lambda i: (i, 0)),
            pl.BlockSpec((2 * BM, HALF), lambda i: (i, 0)),
            pl.BlockSpec((BM, BM), lambda i: (0, 0)),
            pl.BlockSpec((NBM, BM), lambda i: (0, 0)),
            pl.BlockSpec((NBM, BM), lambda i: (0, 0)),
            pl.BlockSpec((1, C_OUT), lambda i: (0, 0)),
        ],
        out_specs=pl.BlockSpec((BM, C_OUT), lambda i: (i, 0)),
        out_shape=jax.ShapeDtypeStruct((N, C_OUT), jnp.float32),
    )


def kernel(x, edge_index, W, b):
    ei = edge_index.astype(jnp.int32)
    src = ei[0]
    dst = ei[1]
    blk = src // BM
    off = src % BM
    pad_i = jnp.full((EPAD - E,), NC * N, jnp.int32)
    pad_d = jnp.zeros((EPAD - E,), jnp.int32)
    s0 = blk * (2 * BM) + off
    src2d = jnp.concatenate([s0, pad_i, s0 + BM, pad_i]).reshape(2 * ERC, B2)
    dst2d = jnp.concatenate([dst, pad_d]).reshape(ERC, B1)
    z128 = jnp.zeros((N, HALF), jnp.float32)

    eye = jnp.eye(BM, dtype=jnp.float32)

    degp = _make_deg()(dst2d)
    d0 = degp[:N].reshape(NBM, BM)
    d1 = degp[NP:NP + N].reshape(NBM, BM)
    hs2 = _make_mm()(x, W, eye, d0, d1)
    acc2 = _make_agg()(src2d, dst2d, hs2, z128)
    return _make_fin()(acc2, hs2, eye, d0, d1, b.reshape(1, C_OUT))


# speedup vs baseline: 1.9229x; 1.9229x over previous
import jax
import jax.numpy as jnp
from jax import lax
from jax.experimental import pallas as pl
from jax.experimental.pallas import tpu as pltpu
from jax.experimental.pallas import tpu_sc as plsc

N = 10000
E = 160000
C_IN = 256
C_OUT = 256
HALF = 128
NS = 16
NC = 2
CH = 624
TAIL = N - NS * CH

NP = 10240
CH1 = NP // NS

B1 = 128
EPW = 4992
NB1 = EPW // B1
REM = E - NC * NS * EPW
REMB = REM // B1

B2 = 128
EPT = 9984
NB2 = EPT // B2
REM2 = E - NS * EPT
REMB2 = REM2 // B2


def _make_mesh():
    return plsc.VectorSubcoreMesh(
        core_axis_name="c", subcore_axis_name="s", num_cores=NC, num_subcores=NS
    )


def _deg_body(dst_hbm, degp_hbm, dacc, idx_v, ones_v, zbuf):
    c = lax.axis_index("c")
    s = lax.axis_index("s")
    ones = jnp.ones((16,), jnp.float32)
    zeros = jnp.zeros((16,), jnp.float32)

    @pl.loop(0, B1 // 16)
    def _(i):
        ones_v[pl.ds(i * 16, 16)] = ones

    @pl.loop(0, CH1 // 16)
    def _(i):
        zbuf[pl.ds(i * 16, 16)] = zeros

    r0 = s * CH1
    pltpu.sync_copy(zbuf, dacc.at[pl.ds(r0, CH1)])
    plsc.subcore_barrier()
    base = (c * NS + s) * EPW

    @pl.loop(0, NB1)
    def _(g):
        pltpu.sync_copy(dst_hbm.at[pl.ds(base + g * B1, B1)], idx_v)
        pltpu.sync_copy(ones_v, dacc.at[idx_v], add=True)

    @pl.when((c == 0) & (s == 0))
    def _():
        @pl.loop(0, REMB)
        def _(g):
            pltpu.sync_copy(dst_hbm.at[pl.ds(NC * NS * EPW + g * B1, B1)], idx_v)
            pltpu.sync_copy(ones_v, dacc.at[idx_v], add=True)

    plsc.subcore_barrier()
    pltpu.sync_copy(dacc.at[pl.ds(r0, CH1)], degp_hbm.at[pl.ds(c * NP + r0, CH1)])


def _make_deg():
    return pl.kernel(
        _deg_body,
        out_type=jax.ShapeDtypeStruct((NC * NP,), jnp.float32),
        mesh=_make_mesh(),
        scratch_types=[
            pltpu.VMEM_SHARED((NP,), jnp.float32),
            pltpu.VMEM((B1,), jnp.int32),
            pltpu.VMEM((B1,), jnp.float32),
            pltpu.VMEM((CH1,), jnp.float32),
        ],
    )


def _agg_body(src2_hbm, dst_hbm, hs2_hbm, z128_hbm, acc2_hbm, asp,
              sidx0, sidx1, didx0, didx1, rows0, rows1, sem0, sem1, ssem0, ssem1):
    c = lax.axis_index("c")
    s = lax.axis_index("s")
    r0 = s * CH
    pltpu.sync_copy(z128_hbm.at[pl.ds(r0, CH)], asp.at[pl.ds(r0, CH)])

    @pl.when(s == 0)
    def _():
        pltpu.sync_copy(z128_hbm.at[pl.ds(NS * CH, TAIL)], asp.at[pl.ds(NS * CH, TAIL)])

    plsc.subcore_barrier()
    eb = s * EPT
    sb = c * E + eb
    sidx = (sidx0, sidx1)
    didx = (didx0, didx1)
    rows = (rows0, rows1)
    sems = (sem0, sem1)
    ssems = (ssem0, ssem1)

    pltpu.sync_copy(src2_hbm.at[pl.ds(sb, B2)], sidx0)
    pltpu.sync_copy(dst_hbm.at[pl.ds(eb, B2)], didx0)
    pltpu.async_copy(hs2_hbm.at[sidx0], rows0, sem0)

    @pl.loop(0, NB2 // 2)
    def _(h):
        for b in range(2):
            g = 2 * h + b
            nxt = g + 1
            nb = 1 - b

            @pl.when(nxt < NB2)
            def _():
                @pl.when(g >= 1)
                def _():
                    pltpu.make_async_copy(rows[nb], asp.at[didx[nb]], ssems[nb]).wait()

                pltpu.sync_copy(src2_hbm.at[pl.ds(sb + nxt * B2, B2)], sidx[nb])
                pltpu.sync_copy(dst_hbm.at[pl.ds(eb + nxt * B2, B2)], didx[nb])
                pltpu.async_copy(hs2_hbm.at[sidx[nb]], rows[nb], sems[nb])

            pltpu.make_async_copy(hs2_hbm.at[sidx[b]], rows[b], sems[b]).wait()
            pltpu.async_copy(rows[b], asp.at[didx[b]], ssems[b], add=True)

    pltpu.make_async_copy(rows0, asp.at[didx0], ssem0).wait()
    pltpu.make_async_copy(rows1, asp.at[didx1], ssem1).wait()

    @pl.when(s == 0)
    def _():
        @pl.loop(0, REMB2)
        def _(g):
            pltpu.sync_copy(src2_hbm.at[pl.ds(c * E + NS * EPT + g * B2, B2)], sidx0)
            pltpu.sync_copy(dst_hbm.at[pl.ds(NS * EPT + g * B2, B2)], didx0)
            pltpu.async_copy(hs2_hbm.at[sidx0], rows0, sem0).wait()
            pltpu.sync_copy(rows0, asp.at[didx0], add=True)

    plsc.subcore_barrier()
    for rep in range(2):
        g = s + NS * rep

        @pl.when(g < NBM)
        def _():
            pltpu.sync_copy(
                asp.at[pl.ds(g * BM, BM)],
                acc2_hbm.at[pl.ds(g * 2 * BM + c * BM, BM)],
            )


def _make_agg():
    return pl.kernel(
        _agg_body,
        out_type=jax.ShapeDtypeStruct((NC * N, HALF), jnp.float32),
        mesh=_make_mesh(),
        scratch_types=[
            pltpu.VMEM_SHARED((N, HALF), jnp.float32),
            pltpu.VMEM((B2,), jnp.int32),
            pltpu.VMEM((B2,), jnp.int32),
            pltpu.VMEM((B2,), jnp.int32),
            pltpu.VMEM((B2,), jnp.int32),
            pltpu.VMEM((B2, HALF), jnp.float32),
            pltpu.VMEM((B2, HALF), jnp.float32),
            pltpu.SemaphoreType.DMA,
            pltpu.SemaphoreType.DMA,
            pltpu.SemaphoreType.DMA,
            pltpu.SemaphoreType.DMA,
        ],
    )


BM = 400
NBM = N // BM


def _dinv_col(eye_b, d0_b, d1_b, i):
    d0 = d0_b[pl.ds(i, 1), :]
    d1 = d1_b[pl.ds(i, 1), :]
    dinv_row = lax.rsqrt(1.0 + d0 + d1)
    return lax.dot_general(eye_b[...], dinv_row, (((1,), (1,)), ((), ())),
                           preferred_element_type=jnp.float32)


def _mm_body(x_b, w_b, eye_b, d0_b, d1_b, o_b):
    dinv = _dinv_col(eye_b, d0_b, d1_b, pl.program_id(0))
    h = jnp.dot(x_b[...], w_b[...], preferred_element_type=jnp.float32)
    o_b[pl.ds(0, BM), :] = h[:, :HALF] * dinv
    o_b[pl.ds(BM, BM), :] = h[:, HALF:] * dinv


def _make_mm():
    return pl.pallas_call(
        _mm_body,
        grid=(NBM,),
        in_specs=[
            pl.BlockSpec((BM, C_IN), lambda i: (i, 0)),
            pl.BlockSpec((C_IN, C_OUT), lambda i: (0, 0)),
            pl.BlockSpec((BM, BM), lambda i: (0, 0)),
            pl.BlockSpec((NBM, BM), lambda i: (0, 0)),
            pl.BlockSpec((NBM, BM), lambda i: (0, 0)),
        ],
        out_specs=pl.BlockSpec((2 * BM, HALF), lambda i: (i, 0)),
        out_shape=jax.ShapeDtypeStruct((NC * N, HALF), jnp.float32),
    )


def _fin_body(a_b, h_b, eye_b, d0_b, d1_b, b_b, o_b):
    dinv = _dinv_col(eye_b, d0_b, d1_b, pl.program_id(0))
    t0 = (a_b[pl.ds(0, BM), :] + h_b[pl.ds(0, BM), :]) * dinv
    t1 = (a_b[pl.ds(BM, BM), :] + h_b[pl.ds(BM, BM), :]) * dinv
    o_b[...] = jnp.maximum(jnp.concatenate([t0, t1], axis=1) + b_b[...], 0.0)


def _make_fin():
    return pl.pallas_call(
        _fin_body,
        grid=(NBM,),
        in_specs=[
            pl.BlockSpec((2 * BM, HALF), lambda i: (i, 0)),
            pl.BlockSpec((2 * BM, HALF), lambda i: (i, 0)),
            pl.BlockSpec((BM, BM), lambda i: (0, 0)),
            pl.BlockSpec((NBM, BM), lambda i: (0, 0)),
            pl.BlockSpec((NBM, BM), lambda i: (0, 0)),
            pl.BlockSpec((1, C_OUT), lambda i: (0, 0)),
        ],
        out_specs=pl.BlockSpec((BM, C_OUT), lambda i: (i, 0)),
        out_shape=jax.ShapeDtypeStruct((N, C_OUT), jnp.float32),
    )


def kernel(x, edge_index, W, b):
    ei = edge_index.astype(jnp.int32)
    src = ei[0]
    dst = ei[1]
    blk = src // BM
    off = src % BM
    src2 = jnp.concatenate([blk * (2 * BM) + off, blk * (2 * BM) + BM + off])
    z128 = jnp.zeros((N, HALF), jnp.float32)
    eye = jnp.eye(BM, dtype=jnp.float32)

    degp = _make_deg()(dst)
    d0 = degp[:N].reshape(NBM, BM)
    d1 = degp[NP:NP + N].reshape(NBM, BM)
    hs2 = _make_mm()(x, W, eye, d0, d1)
    acc2 = _make_agg()(src2, dst, hs2, z128)
    return _make_fin()(acc2, hs2, eye, d0, d1, b.reshape(1, C_OUT))
